# baseline (speedup 1.0000x reference)
import functools

import jax
import jax.numpy as jnp
from jax import lax
from jax.experimental import pallas as pl
from jax.experimental.pallas import tpu as pltpu
from jax.experimental.pallas import tpu_sc as plsc

_B, _S = 4096, 50
_D = 32
_B_TOTAL = _B * _S
_NC, _NS = 2, 16
_NW = _NC * _NS
_B_PER_W = _B_TOTAL // _NW
_GW = 800
_N_STREAMS = _B_PER_W // _GW


@jax.jit
def _gather_rows(table, idx):
    mesh = plsc.VectorSubcoreMesh(core_axis_name="c", subcore_axis_name="s")

    @functools.partial(
        pl.kernel,
        mesh=mesh,
        out_type=jax.ShapeDtypeStruct((_B_TOTAL, _D), table.dtype),
        scratch_types=[
            pltpu.VMEM((_N_STREAMS, _GW), jnp.int32),
            pltpu.VMEM((_GW, _D), table.dtype),
            pltpu.VMEM((_GW, _D), table.dtype),
            pltpu.SemaphoreType.DMA,
            pltpu.SemaphoreType.DMA,
        ],
        compiler_params=pltpu.CompilerParams(use_tc_tiling_on_sc=False),
    )
    def k(table_hbm, idx_hbm, out_hbm, idx_v, buf0, buf1, sem0, sem1):
        wid = lax.axis_index("s") * _NC + lax.axis_index("c")
        base = wid * _B_PER_W
        pltpu.sync_copy(idx_hbm.at[wid], idx_v)

        bufs = (buf0, buf1)
        sems = (sem0, sem1)

        def fire(i):
            return pltpu.async_copy(
                table_hbm.at[idx_v.at[i]], bufs[i % 2], sems[i % 2]
            )

        handles = [None] * _N_STREAMS
        handles[0] = fire(0)
        for i in range(_N_STREAMS):
            handles[i].wait()
            if i + 1 < _N_STREAMS:
                handles[i + 1] = fire(i + 1)
            pltpu.sync_copy(bufs[i % 2], out_hbm.at[pl.ds(base + i * _GW, _GW)])

    return k(table, idx)


def kernel(x, table):
    idx = x.reshape(_NW, _N_STREAMS, _GW)
    out = _gather_rows(table, idx)
    return out.reshape(_B, _S, _D)


# speedup vs baseline: 1.1575x; 1.1575x over previous
import functools

import jax
import jax.numpy as jnp
from jax import lax
from jax.experimental import pallas as pl
from jax.experimental.pallas import tpu as pltpu
from jax.experimental.pallas import tpu_sc as plsc

_B, _S = 4096, 50
_D = 32
_NC, _NS = 2, 16
_NW = _NC * _NS
_BB = _B // _NW


@jax.jit
def _embed(table, xt):
    mesh = plsc.VectorSubcoreMesh(core_axis_name="c", subcore_axis_name="s")

    @functools.partial(
        pl.kernel,
        mesh=mesh,
        out_type=jax.ShapeDtypeStruct((_S, 4, _NW, 8, _BB), jnp.float32),
        scratch_types=[
            pltpu.VMEM((_S, _BB), jnp.int32),
            pltpu.VMEM((_BB, _D), jnp.float32),
            pltpu.VMEM((_BB, _D), jnp.float32),
            pltpu.VMEM((4, 8, _BB), jnp.float32),
            pltpu.VMEM((4, 8, _BB), jnp.float32),
            pltpu.SemaphoreType.DMA,
            pltpu.SemaphoreType.DMA,
            pltpu.SemaphoreType.DMA,
            pltpu.SemaphoreType.DMA,
        ],
        compiler_params=pltpu.CompilerParams(
            use_tc_tiling_on_sc=False, needs_layout_passes=False
        ),
    )
    def k(tbl, xt_hbm, out, idx_v, g0, g1, t0, t1, gs0, gs1, ss0, ss1):
        w = lax.axis_index("s") * _NC + lax.axis_index("c")
        pltpu.sync_copy(xt_hbm.at[:, pl.ds(w * _BB, _BB)], idx_v)

        gbuf, tbuf = (g0, g1), (t0, t1)
        gsem, ssem = (gs0, gs1), (ss0, ss1)
        rowvecs = [lax.iota(jnp.int32, 16) + 16 * j for j in range(8)]

        def fire(s, par):
            pltpu.async_copy(tbl.at[idx_v.at[s]], gbuf[par], gsem[par])

        def drain_gather(par):
            pltpu.make_async_copy(
                tbl.at[pl.ds(0, _BB)], gbuf[par], gsem[par]
            ).wait()

        def drain_store(par):
            pltpu.make_async_copy(out.at[0, :, w], tbuf[par], ssem[par]).wait()

        def transpose(par):
            g, t = gbuf[par], tbuf[par]
            for tr in range(4):
                for ir in range(8):
                    col = jnp.full((16,), tr * 8 + ir, jnp.int32)
                    for j in range(8):
                        vals = plsc.load_gather(g, [rowvecs[j], col])
                        t[tr, ir, pl.ds(16 * j, 16)] = vals

        def handle(i, s, par):
            drain_gather(par)

            @pl.when(s + 1 < _S)
            def _():
                fire(s + 1, 1 - par)

            @pl.when(i > 0)
            def _():
                drain_store(par)

            transpose(par)
            pltpu.async_copy(tbuf[par], out.at[s, :, w], ssem[par])

        fire(0, 0)

        def body(i, carry):
            handle(i, 2 * i, 0)
            handle(i, 2 * i + 1, 1)
            return carry

        lax.fori_loop(0, _S // 2, body, 0)
        drain_store(0)
        drain_store(1)

    return k(table, xt)


def kernel(x, table):
    xt = jnp.transpose(x)
    lout = _embed(table, xt)
    return jnp.transpose(lout, (2, 4, 0, 1, 3)).reshape(_B, _S, _D)


# speedup vs baseline: 1.1688x; 1.0097x over previous
import functools

import jax
import jax.numpy as jnp
from jax import lax
from jax.experimental import pallas as pl
from jax.experimental.pallas import tpu as pltpu
from jax.experimental.pallas import tpu_sc as plsc

_B, _S = 4096, 50
_D = 32
_NC, _NS = 2, 16
_NW = _NC * _NS
_BB = _B // _NW


@jax.jit
def _embed(t128, xt):
    mesh = plsc.VectorSubcoreMesh(core_axis_name="c", subcore_axis_name="s")

    @functools.partial(
        pl.kernel,
        mesh=mesh,
        out_type=jax.ShapeDtypeStruct((_S, _D, _B), jnp.float32),
        scratch_types=[
            pltpu.VMEM((_S, _BB), jnp.int32),
            pltpu.VMEM((_S, _BB), jnp.int32),
            pltpu.VMEM((_BB, 128), jnp.float32),
            pltpu.VMEM((_BB, 128), jnp.float32),
            pltpu.VMEM((_D, _BB), jnp.float32),
            pltpu.VMEM((_D, _BB), jnp.float32),
            pltpu.SemaphoreType.DMA,
            pltpu.SemaphoreType.DMA,
            pltpu.SemaphoreType.DMA,
            pltpu.SemaphoreType.DMA,
        ],
        compiler_params=pltpu.CompilerParams(
            use_tc_tiling_on_sc=True, needs_layout_passes=False
        ),
    )
    def k(tbl, xt_hbm, out, idx_v, jv, g0, g1, t0, t1, gs0, gs1, ss0, ss1):
        w = lax.axis_index("s") * _NC + lax.axis_index("c")
        pltpu.sync_copy(xt_hbm.at[:, pl.ds(w * _BB, _BB)], idx_v)

        gbuf, tbuf = (g0, g1), (t0, t1)
        gsem, ssem = (gs0, gs1), (ss0, ss1)
        lanes = lax.iota(jnp.int32, 16)
        rowvec = [lanes + 16 * p for p in range(8)]

        def prep(s, carry):
            for p in range(8):
                iv = idx_v[s, pl.ds(16 * p, 16)]
                jv[s, pl.ds(16 * p, 16)] = lax.shift_right_logical(iv, 2)
            return carry

        lax.fori_loop(0, _S, prep, 0)

        def fire(s, par):
            pltpu.async_copy(tbl.at[jv.at[s]], gbuf[par], gsem[par])

        def drain_gather(par):
            pltpu.make_async_copy(
                tbl.at[pl.ds(0, _BB)], gbuf[par], gsem[par]
            ).wait()

        def drain_store(par):
            pltpu.make_async_copy(
                out.at[0, :, pl.ds(0, _BB)], tbuf[par], ssem[par]
            ).wait()

        def transpose_select(s, par):
            g, t = gbuf[par], tbuf[par]
            q32 = []
            for p in range(8):
                iv = idx_v[s, pl.ds(16 * p, 16)]
                q32.append(lax.shift_left(jnp.bitwise_and(iv, 3), 5))
            for m in range(2 * _BB):
                d, p = m // 8, m % 8
                vals = plsc.load_gather(g, [rowvec[p], q32[p] + d])
                t[d, pl.ds(16 * p, 16)] = vals

        def handle(i, s, par):
            drain_gather(par)

            @pl.when(s + 1 < _S)
            def _():
                fire(s + 1, 1 - par)

            @pl.when(i > 0)
            def _():
                drain_store(par)

            transpose_select(s, par)
            pltpu.async_copy(
                tbuf[par], out.at[s, :, pl.ds(w * _BB, _BB)], ssem[par]
            )

        fire(0, 0)

        def body(i, carry):
            handle(i, 2 * i, 0)
            handle(i, 2 * i + 1, 1)
            return carry

        lax.fori_loop(0, _S // 2, body, 0)
        drain_store(0)
        drain_store(1)

    return k(t128, xt)


def kernel(x, table):
    t128 = table.reshape(_D * 1000000 // 128, 128)
    xt = jnp.transpose(x)
    out = _embed(t128, xt)
    return jnp.transpose(out, (2, 0, 1))


# speedup vs baseline: 1.2411x; 1.0619x over previous
import functools

import jax
import jax.numpy as jnp
from jax import lax
from jax.experimental import pallas as pl
from jax.experimental.pallas import tpu as pltpu
from jax.experimental.pallas import tpu_sc as plsc

_B, _S = 4096, 50
_D = 32
_NC, _NS = 2, 16
_NW = _NC * _NS
_BB = _B // _NW


@jax.jit
def _embed(t128, xt):
    mesh = plsc.VectorSubcoreMesh(core_axis_name="c", subcore_axis_name="s")

    @functools.partial(
        pl.kernel,
        mesh=mesh,
        out_type=jax.ShapeDtypeStruct((_S, _D, _B), jnp.float32),
        scratch_types=[
            pltpu.VMEM((_S, _BB), jnp.int32),
            pltpu.VMEM((_S, _BB), jnp.int32),
            pltpu.VMEM((_BB, 128), jnp.float32),
            pltpu.VMEM((_BB, 128), jnp.float32),
            pltpu.VMEM((_D, _BB), jnp.float32),
            pltpu.VMEM((_D, _BB), jnp.float32),
            pltpu.SemaphoreType.DMA,
            pltpu.SemaphoreType.DMA,
            pltpu.SemaphoreType.DMA,
            pltpu.SemaphoreType.DMA,
        ],
        compiler_params=pltpu.CompilerParams(
            use_tc_tiling_on_sc=True, needs_layout_passes=False
        ),
    )
    def k(tbl, xt_hbm, out, idx_v, jv, g0, g1, t0, t1, gs0, gs1, ss0, ss1):
        w = lax.axis_index("s") * _NC + lax.axis_index("c")
        pltpu.sync_copy(xt_hbm.at[:, pl.ds(w * _BB, _BB)], idx_v)

        gbuf, tbuf = (g0, g1), (t0, t1)
        gsem, ssem = (gs0, gs1), (ss0, ss1)
        lanes = lax.iota(jnp.int32, 16)
        rowvec = [lanes + 16 * p for p in range(8)]

        def prep(s, carry):
            for p in range(8):
                iv = idx_v[s, pl.ds(16 * p, 16)]
                jv[s, pl.ds(16 * p, 16)] = lax.shift_right_logical(iv, 2)
            return carry

        lax.fori_loop(0, _S, prep, 0)

        def fire(s, par):
            pltpu.async_copy(tbl.at[jv.at[s]], gbuf[par], gsem[par])

        def drain_gather(par):
            pltpu.make_async_copy(
                tbl.at[pl.ds(0, _BB)], gbuf[par], gsem[par]
            ).wait()

        def drain_store(par):
            pltpu.make_async_copy(
                out.at[0, :, pl.ds(0, _BB)], tbuf[par], ssem[par]
            ).wait()

        def transpose_select(s, par):
            g, t = gbuf[par], tbuf[par]
            q32 = []
            for p in range(8):
                iv = idx_v[s, pl.ds(16 * p, 16)]
                q32.append(lax.shift_left(jnp.bitwise_and(iv, 3), 5))
            for m0 in range(0, 2 * _BB, 8):
                vals = []
                for m in range(m0, m0 + 8):
                    d, p = m // 8, m % 8
                    vals.append(plsc.load_gather(g, [rowvec[p], q32[p] + d]))
                for m in range(m0, m0 + 8):
                    d, p = m // 8, m % 8
                    t[d, pl.ds(16 * p, 16)] = vals[m - m0]

        def handle(i, s, par):
            drain_gather(par)

            @pl.when(s + 1 < _S)
            def _():
                fire(s + 1, 1 - par)

            @pl.when(i > 0)
            def _():
                drain_store(par)

            transpose_select(s, par)
            pltpu.async_copy(
                tbuf[par], out.at[s, :, pl.ds(w * _BB, _BB)], ssem[par]
            )

        fire(0, 0)

        def body(i, carry):
            handle(i, 2 * i, 0)
            handle(i, 2 * i + 1, 1)
            return carry

        lax.fori_loop(0, _S // 2, body, 0)
        drain_store(0)
        drain_store(1)

    return k(t128, xt)


def kernel(x, table):
    t128 = table.reshape(_D * 1000000 // 128, 128)
    xt = jnp.transpose(x)
    out = _embed(t128, xt)
    return jnp.transpose(out, (2, 0, 1))
